# baseline (speedup 1.0000x reference)
import functools

import jax
import jax.numpy as jnp
from jax import lax
from jax.experimental import pallas as pl
from jax.experimental.pallas import tpu as pltpu
from jax.experimental.pallas import tpu_sc as plsc

N = 10000
E = 320000
D = 128
EPS = 1e-5

NC = 2
NS = 16
NW = NC * NS
W = 64
NWIN = 160
NROWS = NW * NWIN
EPAD = NROWS * W
NPAD = 10240
RPS = NPAD // NS
NDISC = NPAD - N

@functools.cache
def _mesh():
    return plsc.VectorSubcoreMesh(core_axis_name="c", subcore_axis_name="s")


def _sc_agg_call(z, src2, dst2, with_deg):
    out_type = [jax.ShapeDtypeStruct((NC, NPAD, D), jnp.float32)]
    if with_deg:
        out_type.append(jax.ShapeDtypeStruct((NC, NPAD), jnp.float32))

    NB = 4
    LEAD = 2
    ICH = 16
    NCH = NWIN // ICH
    scratch_types = [
        pltpu.VMEM((2, ICH, W), jnp.int32),
        pltpu.VMEM((2, ICH, W), jnp.int32),
        pltpu.VMEM((1, W), jnp.float32),
        pltpu.VMEM((RPS,), jnp.float32),
        pltpu.VMEM_SHARED((NPAD, D), jnp.float32),
        pltpu.VMEM_SHARED((NPAD,), jnp.float32),
    ]
    scratch_types += [pltpu.VMEM((W, D), jnp.float32)] * NB
    scratch_types += [pltpu.SemaphoreType.DMA] * (2 * NB + 2)

    def body(*refs):
        n_in = 3
        n_out = 2 if with_deg else 1
        z_hbm, src_hbm, dst_hbm = refs[:n_in]
        agg_out = refs[n_in]
        deg_out = refs[n_in + 1] if with_deg else None
        sc = refs[n_in + n_out:]
        srcv, dstv, onesv, dzv, agg_sh, deg_sh = sc[:6]
        bufs = sc[6:6 + NB]
        gsems = sc[6 + NB:6 + 2 * NB]
        ssems = sc[6 + 2 * NB:6 + 3 * NB]
        isems = sc[6 + 3 * NB:6 + 3 * NB + 2]
        cid = lax.axis_index("c")
        sid = lax.axis_index("s")
        wid = sid * NC + cid

        def stage_idx(chunk, slot):
            base = wid * NWIN + chunk * ICH
            pltpu.async_copy(src_hbm.at[pl.ds(base, ICH)],
                             srcv.at[slot], isems[slot])
            pltpu.async_copy(dst_hbm.at[pl.ds(base, ICH)],
                             dstv.at[slot], isems[slot])

        def wait_idx(slot):
            pltpu.make_async_copy(src_hbm.at[pl.ds(0, ICH)],
                                  srcv.at[slot], isems[slot]).wait()
            pltpu.make_async_copy(dst_hbm.at[pl.ds(0, ICH)],
                                  dstv.at[slot], isems[slot]).wait()

        stage_idx(0, 0)
        stage_idx(1, 1)

        @pl.loop(0, W)
        def _(i):
            for k in range(D // 16):
                bufs[0][i, pl.ds(k * 16, 16)] = jnp.zeros((16,), jnp.float32)

        @pl.loop(0, W, step=16)
        def _(i):
            onesv[0, pl.ds(i, 16)] = jnp.ones((16,), jnp.float32)

        @pl.loop(0, RPS, step=W)
        def _(i):
            pltpu.sync_copy(bufs[0], agg_sh.at[pl.ds(sid * RPS + i, W)])
        if with_deg:
            @pl.loop(0, RPS, step=16)
            def _(i):
                dzv[pl.ds(i, 16)] = jnp.zeros((16,), jnp.float32)
            pltpu.sync_copy(dzv, deg_sh.at[pl.ds(sid * RPS, RPS)])

        plsc.subcore_barrier()

        for chunk in range(NCH):
            slot = chunk % 2
            wait_idx(slot)
            for v in range(NB):
                pltpu.async_copy(z_hbm.at[srcv.at[slot].at[v]],
                                 bufs[v], gsems[v])

            @pl.loop(0, ICH, step=NB)
            def _(w):
                for b in range(NB):
                    ww = w + b
                    pltpu.make_async_copy(z_hbm.at[srcv.at[slot].at[ww]],
                                          bufs[b], gsems[b]).wait()
                    if with_deg:
                        pltpu.sync_copy(onesv.at[0],
                                        deg_sh.at[dstv.at[slot].at[ww]],
                                        add=True)
                    sdesc = pltpu.async_copy(bufs[b],
                                             agg_sh.at[dstv.at[slot].at[ww]],
                                             ssems[b], add=True)

                    @pl.when(ww + NB < ICH)
                    def _():
                        sdesc.wait()
                        pltpu.async_copy(z_hbm.at[srcv.at[slot].at[ww + NB]],
                                         bufs[b], gsems[b])

            for b in range(NB):
                pltpu.make_async_copy(bufs[b],
                                      agg_sh.at[dstv.at[slot].at[ICH - NB + b]],
                                      ssems[b]).wait()
            if chunk + 2 < NCH:
                stage_idx(chunk + 2, slot)

        plsc.subcore_barrier()

        pltpu.sync_copy(agg_sh.at[pl.ds(sid * RPS, RPS)],
                        agg_out.at[cid, pl.ds(sid * RPS, RPS)])
        if with_deg:
            pltpu.sync_copy(deg_sh.at[pl.ds(sid * RPS, RPS)],
                            deg_out.at[cid, pl.ds(sid * RPS, RPS)])

    return pl.kernel(body, out_type=out_type, mesh=_mesh(),
                     scratch_types=scratch_types)(z, src2, dst2)


BLK = 5000
NBLK = N // BLK

_SH = jax.ShapeDtypeStruct((N, D), jnp.float32)


def _rowspec(bs=BLK):
    return pl.BlockSpec((bs, D), lambda i: (i, 0))


def _wspec(shape):
    nd = len(shape)
    return pl.BlockSpec(shape, lambda i: (0,) * nd)


def _matmul_t(a, w):
    return lax.dot_general(a, w, (((1,), (1,)), ((), ())),
                           precision=lax.Precision.HIGHEST,
                           preferred_element_type=jnp.float32)


def _tc_pre(x, S0, S1):
    def body(x_ref, s0_ref, s1_ref, xs0_ref, xs1_ref):
        x_ = x_ref[...]
        xs0_ref[...] = _matmul_t(x_, s0_ref[...])
        xs1_ref[...] = _matmul_t(x_, s1_ref[...])

    return pl.pallas_call(
        body,
        grid=(NBLK,),
        in_specs=[_rowspec(), _wspec((D, D)), _wspec((D, D))],
        out_specs=[_rowspec(), _rowspec()],
        out_shape=[_SH, _SH],
    )(x, S0, S1)


def _tc_recip(degp):
    def body(dg_ref, recip_ref):
        deg = dg_ref[0, :N] + dg_ref[1, :N]
        recip_ref[...] = (1.0 / jnp.maximum(deg, 1.0))[:, None]

    return pl.pallas_call(
        body, out_shape=jax.ShapeDtypeStruct((N, 1), jnp.float32))(degp)


def _tc_layer(p, recip, z, Wl, bl, Wr, lnw, lnb, a, skips, n_out):
    nskip = len(skips)

    def body(*refs):
        (p_ref, r_ref, z_ref, wl_ref, bl_ref, wr_ref,
         lnw_ref, lnb_ref, a_ref) = refs[:9]
        skip_refs = refs[9:9 + nskip]
        out_refs = refs[9 + nskip:9 + nskip + n_out]
        acc_ref, hs_ref = refs[9 + nskip + n_out:]
        i = pl.program_id(0)

        @pl.when(i < NBLK)
        def _():
            agg = (p_ref[0] + p_ref[1]) * r_ref[...]
            h = (_matmul_t(agg, wl_ref[...]) + bl_ref[...][None, :]
                 + _matmul_t(z_ref[...], wr_ref[...]))
            row = pl.multiple_of(i * BLK, BLK)
            hs_ref[pl.ds(row, BLK), :] = h

            @pl.when(i == 0)
            def _():
                acc_ref[0] = 0.0
                acc_ref[1] = 0.0

            acc_ref[0] += jnp.sum(h)
            acc_ref[1] += jnp.sum(h * h)

        @pl.when(i >= NBLK)
        def _():
            cnt = float(N * D)
            m = acc_ref[0] / cnt
            var = acc_ref[1] / cnt - m * m
            std = jnp.sqrt(jnp.maximum(var, 0.0))
            row = pl.multiple_of((i - NBLK) * BLK, BLK)
            h = hs_ref[pl.ds(row, BLK), :]
            hn = ((h - m) / (std + EPS)) * lnw_ref[...][None, :] \
                + lnb_ref[...][None, :]
            act = jnp.where(hn >= 0.0, hn, a_ref[0] * hn)
            if n_out == 2:
                out_refs[0][...] = act
                out_refs[1][...] = act + skip_refs[0][...]
            elif nskip == 2:
                out_refs[0][...] = act + skip_refs[0][...] + skip_refs[1][...]
            else:
                out_refs[0][...] = act

    def _a_idx(i):
        return jnp.minimum(i, NBLK - 1)

    def _b_idx(i):
        return jnp.maximum(i - NBLK, 0)

    in_specs = [
        pl.BlockSpec((NC, BLK, D), lambda i: (0, _a_idx(i), 0)),
        pl.BlockSpec((BLK, 1), lambda i: (_a_idx(i), 0)),
        pl.BlockSpec((BLK, D), lambda i: (_a_idx(i), 0)),
        _wspec((D, D)), _wspec((D,)), _wspec((D, D)),
        _wspec((D,)), _wspec((D,)),
        pl.BlockSpec(memory_space=pltpu.SMEM),
    ]
    in_specs += [pl.BlockSpec((BLK, D), lambda i: (_b_idx(i), 0))] * nskip
    return pl.pallas_call(
        body,
        grid=(2 * NBLK,),
        in_specs=in_specs,
        out_specs=[pl.BlockSpec((BLK, D), lambda i: (_b_idx(i), 0))] * n_out,
        out_shape=[_SH] * n_out,
        scratch_shapes=[pltpu.SMEM((2,), jnp.float32),
                        pltpu.VMEM((N, D), jnp.float32)],
    )(p, recip, z, Wl, bl, Wr, lnw, lnb, a, *skips)


def kernel(x, edge_index, Wl0, bl0, Wr0, Wl1, bl1, Wr1, Wl2, bl2, Wr2,
           S0, S1, ln0_w, ln0_b, ln1_w, ln1_b, ln2_w, ln2_b, a0, a1, a2):
    npad_e = EPAD - E
    pad_src = jnp.arange(npad_e, dtype=jnp.int32) % NDISC
    pad_dst = N + pad_src
    src2 = jnp.concatenate([edge_index[0], pad_src]).reshape(NROWS, W)
    dst2 = jnp.concatenate([edge_index[1], pad_dst]).reshape(NROWS, W)

    xs0, xs1 = _tc_pre(x, S0, S1)

    p, degp = _sc_agg_call(x, src2, dst2, with_deg=True)
    recip = _tc_recip(degp)
    h1, z2 = _tc_layer(p, recip, x, Wl0, bl0, Wr0, ln0_w, ln0_b, a0,
                       [xs0], n_out=2)

    (q,) = _sc_agg_call(z2, src2, dst2, with_deg=False)
    (z3,) = _tc_layer(q, recip, z2, Wl1, bl1, Wr1, ln1_w, ln1_b, a1,
                      [h1, xs1], n_out=1)

    (r,) = _sc_agg_call(z3, src2, dst2, with_deg=False)
    (ret,) = _tc_layer(r, recip, z3, Wl2, bl2, Wr2, ln2_w, ln2_b, a2,
                       [], n_out=1)
    return ret


# speedup vs baseline: 1.0644x; 1.0644x over previous
import functools

import jax
import jax.numpy as jnp
from jax import lax
from jax.experimental import pallas as pl
from jax.experimental.pallas import tpu as pltpu
from jax.experimental.pallas import tpu_sc as plsc

N = 10000
E = 320000
D = 128
EPS = 1e-5

NC = 2
NS = 16
NW = NC * NS
W = 64
NWIN = 160
NROWS = NW * NWIN
EPAD = NROWS * W
NPAD = 10240
RPS = NPAD // NS
NDISC = NPAD - N

@functools.cache
def _mesh():
    return plsc.VectorSubcoreMesh(core_axis_name="c", subcore_axis_name="s")


def _sc_agg_call(z, src2, dst2, with_deg):
    out_type = [jax.ShapeDtypeStruct((NC, NPAD, D), jnp.float32)]
    if with_deg:
        out_type.append(jax.ShapeDtypeStruct((NC, NPAD), jnp.float32))

    NB = 4
    LEAD = 2
    ICH = 16
    NCH = NWIN // ICH
    scratch_types = [
        pltpu.VMEM((2, ICH, W), jnp.int32),
        pltpu.VMEM((2, ICH, W), jnp.int32),
        pltpu.VMEM((1, W), jnp.float32),
        pltpu.VMEM((RPS,), jnp.float32),
        pltpu.VMEM_SHARED((NPAD, D), jnp.float32),
        pltpu.VMEM_SHARED((NPAD,), jnp.float32),
    ]
    scratch_types += [pltpu.VMEM((W, D), jnp.float32)] * NB
    scratch_types += [pltpu.SemaphoreType.DMA] * (2 * NB + 2)

    def body(*refs):
        n_in = 3
        n_out = 2 if with_deg else 1
        z_hbm, src_hbm, dst_hbm = refs[:n_in]
        agg_out = refs[n_in]
        deg_out = refs[n_in + 1] if with_deg else None
        sc = refs[n_in + n_out:]
        srcv, dstv, onesv, dzv, agg_sh, deg_sh = sc[:6]
        bufs = sc[6:6 + NB]
        gsems = sc[6 + NB:6 + 2 * NB]
        ssems = sc[6 + 2 * NB:6 + 3 * NB]
        isems = sc[6 + 3 * NB:6 + 3 * NB + 2]
        cid = lax.axis_index("c")
        sid = lax.axis_index("s")
        wid = sid * NC + cid

        def stage_idx(chunk, slot):
            base = wid * NWIN + chunk * ICH
            pltpu.async_copy(src_hbm.at[pl.ds(base, ICH)],
                             srcv.at[slot], isems[slot])
            pltpu.async_copy(dst_hbm.at[pl.ds(base, ICH)],
                             dstv.at[slot], isems[slot])

        def wait_idx(slot):
            pltpu.make_async_copy(src_hbm.at[pl.ds(0, ICH)],
                                  srcv.at[slot], isems[slot]).wait()
            pltpu.make_async_copy(dst_hbm.at[pl.ds(0, ICH)],
                                  dstv.at[slot], isems[slot]).wait()

        stage_idx(0, 0)
        stage_idx(1, 1)

        @pl.loop(0, W)
        def _(i):
            for k in range(D // 16):
                bufs[0][i, pl.ds(k * 16, 16)] = jnp.zeros((16,), jnp.float32)

        @pl.loop(0, W, step=16)
        def _(i):
            onesv[0, pl.ds(i, 16)] = jnp.ones((16,), jnp.float32)

        @pl.loop(0, RPS, step=W)
        def _(i):
            pltpu.sync_copy(bufs[0], agg_sh.at[pl.ds(sid * RPS + i, W)])
        if with_deg:
            @pl.loop(0, RPS, step=16)
            def _(i):
                dzv[pl.ds(i, 16)] = jnp.zeros((16,), jnp.float32)
            pltpu.sync_copy(dzv, deg_sh.at[pl.ds(sid * RPS, RPS)])

        plsc.subcore_barrier()

        for chunk in range(NCH):
            slot = chunk % 2
            wait_idx(slot)
            for v in range(NB):
                pltpu.async_copy(z_hbm.at[srcv.at[slot].at[v]],
                                 bufs[v], gsems[v])

            @pl.loop(0, ICH, step=NB)
            def _(w):
                for b in range(NB):
                    ww = w + b
                    pltpu.make_async_copy(z_hbm.at[srcv.at[slot].at[ww]],
                                          bufs[b], gsems[b]).wait()
                    if with_deg:
                        pltpu.sync_copy(onesv.at[0],
                                        deg_sh.at[dstv.at[slot].at[ww]],
                                        add=True)
                    sdesc = pltpu.async_copy(bufs[b],
                                             agg_sh.at[dstv.at[slot].at[ww]],
                                             ssems[b], add=True)

                    @pl.when(ww + NB < ICH)
                    def _():
                        sdesc.wait()
                        pltpu.async_copy(z_hbm.at[srcv.at[slot].at[ww + NB]],
                                         bufs[b], gsems[b])

            for b in range(NB):
                pltpu.make_async_copy(bufs[b],
                                      agg_sh.at[dstv.at[slot].at[ICH - NB + b]],
                                      ssems[b]).wait()
            if chunk + 2 < NCH:
                stage_idx(chunk + 2, slot)

        plsc.subcore_barrier()

        pltpu.sync_copy(agg_sh.at[pl.ds(sid * RPS, RPS)],
                        agg_out.at[cid, pl.ds(sid * RPS, RPS)])
        if with_deg:
            pltpu.sync_copy(deg_sh.at[pl.ds(sid * RPS, RPS)],
                            deg_out.at[cid, pl.ds(sid * RPS, RPS)])

    return pl.kernel(body, out_type=out_type, mesh=_mesh(),
                     scratch_types=scratch_types)(z, src2, dst2)


BLK = 2000
NBLK = N // BLK

_SH = jax.ShapeDtypeStruct((N, D), jnp.float32)


def _rowspec(bs=BLK):
    return pl.BlockSpec((bs, D), lambda i: (i, 0))


def _wspec(shape):
    nd = len(shape)
    return pl.BlockSpec(shape, lambda i: (0,) * nd)


def _matmul_t(a, w):
    return lax.dot_general(a, w, (((1,), (1,)), ((), ())),
                           precision=lax.Precision.HIGHEST,
                           preferred_element_type=jnp.float32)


def _tc_pre(x, S0, S1, Wr0, bl0):
    def body(x_ref, s0_ref, s1_ref, wr_ref, bl_ref,
             xs0_ref, xs1_ref, w1_ref):
        x_ = x_ref[...]
        xs0_ref[...] = _matmul_t(x_, s0_ref[...])
        xs1_ref[...] = _matmul_t(x_, s1_ref[...])
        w1_ref[...] = _matmul_t(x_, wr_ref[...]) + bl_ref[...][None, :]

    return pl.pallas_call(
        body,
        grid=(NBLK,),
        in_specs=[_rowspec(), _wspec((D, D)), _wspec((D, D)),
                  _wspec((D, D)), _wspec((D,))],
        out_specs=[_rowspec(), _rowspec(), _rowspec()],
        out_shape=[_SH, _SH, _SH],
    )(x, S0, S1, Wr0, bl0)


def _tc_wr(z, Wr, bl):
    def body(z_ref, wr_ref, bl_ref, w_ref):
        w_ref[...] = _matmul_t(z_ref[...], wr_ref[...]) + bl_ref[...][None, :]

    return pl.pallas_call(
        body,
        grid=(NBLK,),
        in_specs=[_rowspec(), _wspec((D, D)), _wspec((D,))],
        out_specs=_rowspec(),
        out_shape=_SH,
    )(z, Wr, bl)


def _tc_recip(degp):
    def body(dg_ref, recip_ref):
        deg = dg_ref[0, :N] + dg_ref[1, :N]
        recip_ref[...] = (1.0 / jnp.maximum(deg, 1.0))[:, None]

    return pl.pallas_call(
        body, out_shape=jax.ShapeDtypeStruct((N, 1), jnp.float32))(degp)


def _tc_layer(p, recip, w, Wl, lnw, lnb, a, skips, n_out):
    nskip = len(skips)

    def body(*refs):
        (p_ref, r_ref, w_ref, wl_ref,
         lnw_ref, lnb_ref, a_ref) = refs[:7]
        skip_refs = refs[7:7 + nskip]
        out_refs = refs[7 + nskip:7 + nskip + n_out]
        acc_ref, hs_ref = refs[7 + nskip + n_out:]
        i = pl.program_id(0)

        @pl.when(i < NBLK)
        def _():
            agg = (p_ref[0] + p_ref[1]) * r_ref[...]
            h = _matmul_t(agg, wl_ref[...]) + w_ref[...]
            row = pl.multiple_of(i * BLK, BLK)
            hs_ref[pl.ds(row, BLK), :] = h

            @pl.when(i == 0)
            def _():
                acc_ref[0] = 0.0
                acc_ref[1] = 0.0

            acc_ref[0] += jnp.sum(h)
            acc_ref[1] += jnp.sum(h * h)

        @pl.when(i >= NBLK)
        def _():
            cnt = float(N * D)
            m = acc_ref[0] / cnt
            var = acc_ref[1] / cnt - m * m
            std = jnp.sqrt(jnp.maximum(var, 0.0))
            row = pl.multiple_of((i - NBLK) * BLK, BLK)
            h = hs_ref[pl.ds(row, BLK), :]
            hn = ((h - m) / (std + EPS)) * lnw_ref[...][None, :] \
                + lnb_ref[...][None, :]
            act = jnp.where(hn >= 0.0, hn, a_ref[0] * hn)
            if n_out == 2:
                out_refs[0][...] = act
                out_refs[1][...] = act + skip_refs[0][...]
            elif nskip == 2:
                out_refs[0][...] = act + skip_refs[0][...] + skip_refs[1][...]
            else:
                out_refs[0][...] = act

    def _a_idx(i):
        return jnp.minimum(i, NBLK - 1)

    def _b_idx(i):
        return jnp.maximum(i - NBLK, 0)

    in_specs = [
        pl.BlockSpec((NC, BLK, D), lambda i: (0, _a_idx(i), 0)),
        pl.BlockSpec((BLK, 1), lambda i: (_a_idx(i), 0)),
        pl.BlockSpec((BLK, D), lambda i: (_a_idx(i), 0)),
        _wspec((D, D)),
        _wspec((D,)), _wspec((D,)),
        pl.BlockSpec(memory_space=pltpu.SMEM),
    ]
    in_specs += [pl.BlockSpec((BLK, D), lambda i: (_b_idx(i), 0))] * nskip
    return pl.pallas_call(
        body,
        grid=(2 * NBLK,),
        in_specs=in_specs,
        out_specs=[pl.BlockSpec((BLK, D), lambda i: (_b_idx(i), 0))] * n_out,
        out_shape=[_SH] * n_out,
        scratch_shapes=[pltpu.SMEM((2,), jnp.float32),
                        pltpu.VMEM((N, D), jnp.float32)],
    )(p, recip, w, Wl, lnw, lnb, a, *skips)


def kernel(x, edge_index, Wl0, bl0, Wr0, Wl1, bl1, Wr1, Wl2, bl2, Wr2,
           S0, S1, ln0_w, ln0_b, ln1_w, ln1_b, ln2_w, ln2_b, a0, a1, a2):
    npad_e = EPAD - E
    pad_src = jnp.arange(npad_e, dtype=jnp.int32) % NDISC
    pad_dst = N + pad_src
    src2 = jnp.concatenate([edge_index[0], pad_src]).reshape(NROWS, W)
    dst2 = jnp.concatenate([edge_index[1], pad_dst]).reshape(NROWS, W)

    xs0, xs1, w1 = _tc_pre(x, S0, S1, Wr0, bl0)

    p, degp = _sc_agg_call(x, src2, dst2, with_deg=True)
    recip = _tc_recip(degp)
    h1, z2 = _tc_layer(p, recip, w1, Wl0, ln0_w, ln0_b, a0,
                       [xs0], n_out=2)

    (q,) = _sc_agg_call(z2, src2, dst2, with_deg=False)
    w2 = _tc_wr(z2, Wr1, bl1)
    (z3,) = _tc_layer(q, recip, w2, Wl1, ln1_w, ln1_b, a1,
                      [h1, xs1], n_out=1)

    (r,) = _sc_agg_call(z3, src2, dst2, with_deg=False)
    w3 = _tc_wr(z3, Wr2, bl2)
    (ret,) = _tc_layer(r, recip, w3, Wl2, ln2_w, ln2_b, a2,
                       [], n_out=1)
    return ret


# speedup vs baseline: 1.0666x; 1.0021x over previous
import functools

import jax
import jax.numpy as jnp
from jax import lax
from jax.experimental import pallas as pl
from jax.experimental.pallas import tpu as pltpu
from jax.experimental.pallas import tpu_sc as plsc

N = 10000
E = 320000
D = 128
EPS = 1e-5

NC = 2
NS = 16
NW = NC * NS
W = 64
NWIN = 160
NROWS = NW * NWIN
EPAD = NROWS * W
NPAD = 10240
RPS = NPAD // NS
NDISC = NPAD - N

@functools.cache
def _mesh():
    return plsc.VectorSubcoreMesh(core_axis_name="c", subcore_axis_name="s")


def _sc_agg_call(z, src2, dst2, with_deg):
    out_type = [jax.ShapeDtypeStruct((NC, NPAD, D), jnp.float32)]
    if with_deg:
        out_type.append(jax.ShapeDtypeStruct((NC, NPAD), jnp.float32))

    NB = 4
    LEAD = 2
    ICH = 16
    NCH = NWIN // ICH
    scratch_types = [
        pltpu.VMEM((2, ICH, W), jnp.int32),
        pltpu.VMEM((2, ICH, W), jnp.int32),
        pltpu.VMEM((1, W), jnp.float32),
        pltpu.VMEM((RPS,), jnp.float32),
        pltpu.VMEM_SHARED((NPAD, D), jnp.float32),
        pltpu.VMEM_SHARED((NPAD,), jnp.float32),
    ]
    scratch_types += [pltpu.VMEM((W, D), jnp.float32)] * NB
    scratch_types += [pltpu.SemaphoreType.DMA] * (2 * NB + 3)

    def body(*refs):
        n_in = 3
        n_out = 2 if with_deg else 1
        z_hbm, src_hbm, dst_hbm = refs[:n_in]
        agg_out = refs[n_in]
        deg_out = refs[n_in + 1] if with_deg else None
        sc = refs[n_in + n_out:]
        srcv, dstv, onesv, dzv, agg_sh, deg_sh = sc[:6]
        bufs = sc[6:6 + NB]
        gsems = sc[6 + NB:6 + 2 * NB]
        ssems = sc[6 + 2 * NB:6 + 3 * NB]
        isems = sc[6 + 3 * NB:6 + 3 * NB + 2]
        zsem = sc[6 + 3 * NB + 2]
        cid = lax.axis_index("c")
        sid = lax.axis_index("s")
        wid = sid * NC + cid

        def stage_idx(chunk, slot):
            base = wid * NWIN + chunk * ICH
            pltpu.async_copy(src_hbm.at[pl.ds(base, ICH)],
                             srcv.at[slot], isems[slot])
            pltpu.async_copy(dst_hbm.at[pl.ds(base, ICH)],
                             dstv.at[slot], isems[slot])

        def wait_idx(slot):
            pltpu.make_async_copy(src_hbm.at[pl.ds(0, ICH)],
                                  srcv.at[slot], isems[slot]).wait()
            pltpu.make_async_copy(dst_hbm.at[pl.ds(0, ICH)],
                                  dstv.at[slot], isems[slot]).wait()

        stage_idx(0, 0)
        stage_idx(1, 1)

        @pl.loop(0, W)
        def _(i):
            for k in range(D // 16):
                bufs[0][i, pl.ds(k * 16, 16)] = jnp.zeros((16,), jnp.float32)

        if with_deg:
            @pl.loop(0, W, step=16)
            def _(i):
                onesv[0, pl.ds(i, 16)] = jnp.ones((16,), jnp.float32)

        for k in range(RPS // W):
            pltpu.async_copy(bufs[0], agg_sh.at[pl.ds(sid * RPS + k * W, W)],
                             zsem)
        if with_deg:
            @pl.loop(0, RPS, step=16)
            def _(i):
                dzv[pl.ds(i, 16)] = jnp.zeros((16,), jnp.float32)
            pltpu.async_copy(dzv, deg_sh.at[pl.ds(sid * RPS, RPS)], zsem)
        for k in range(RPS // W):
            pltpu.make_async_copy(bufs[0],
                                  agg_sh.at[pl.ds(sid * RPS + k * W, W)],
                                  zsem).wait()
        if with_deg:
            pltpu.make_async_copy(dzv, deg_sh.at[pl.ds(sid * RPS, RPS)],
                                  zsem).wait()

        plsc.subcore_barrier()

        for chunk in range(NCH):
            slot = chunk % 2
            wait_idx(slot)
            for v in range(NB):
                pltpu.async_copy(z_hbm.at[srcv.at[slot].at[v]],
                                 bufs[v], gsems[v])

            @pl.loop(0, ICH, step=NB)
            def _(w):
                for b in range(NB):
                    ww = w + b
                    pltpu.make_async_copy(z_hbm.at[srcv.at[slot].at[ww]],
                                          bufs[b], gsems[b]).wait()
                    if with_deg:
                        pltpu.sync_copy(onesv.at[0],
                                        deg_sh.at[dstv.at[slot].at[ww]],
                                        add=True)
                    sdesc = pltpu.async_copy(bufs[b],
                                             agg_sh.at[dstv.at[slot].at[ww]],
                                             ssems[b], add=True)

                    @pl.when(ww + NB < ICH)
                    def _():
                        sdesc.wait()
                        pltpu.async_copy(z_hbm.at[srcv.at[slot].at[ww + NB]],
                                         bufs[b], gsems[b])

            for b in range(NB):
                pltpu.make_async_copy(bufs[b],
                                      agg_sh.at[dstv.at[slot].at[ICH - NB + b]],
                                      ssems[b]).wait()
            if chunk + 2 < NCH:
                stage_idx(chunk + 2, slot)

        plsc.subcore_barrier()

        pltpu.sync_copy(agg_sh.at[pl.ds(sid * RPS, RPS)],
                        agg_out.at[cid, pl.ds(sid * RPS, RPS)])
        if with_deg:
            pltpu.sync_copy(deg_sh.at[pl.ds(sid * RPS, RPS)],
                            deg_out.at[cid, pl.ds(sid * RPS, RPS)])

    return pl.kernel(body, out_type=out_type, mesh=_mesh(),
                     scratch_types=scratch_types)(z, src2, dst2)


BLK = 2000
NBLK = N // BLK

_SH = jax.ShapeDtypeStruct((N, D), jnp.float32)


def _rowspec(bs=BLK):
    return pl.BlockSpec((bs, D), lambda i: (i, 0))


def _wspec(shape):
    nd = len(shape)
    return pl.BlockSpec(shape, lambda i: (0,) * nd)


def _matmul_t(a, w):
    return lax.dot_general(a, w, (((1,), (1,)), ((), ())),
                           precision=lax.Precision.HIGHEST,
                           preferred_element_type=jnp.float32)


def _tc_pre(x, S0, S1, Wr0, bl0):
    def body(x_ref, s0_ref, s1_ref, wr_ref, bl_ref,
             xs0_ref, xs1_ref, w1_ref):
        x_ = x_ref[...]
        xs0_ref[...] = _matmul_t(x_, s0_ref[...])
        xs1_ref[...] = _matmul_t(x_, s1_ref[...])
        w1_ref[...] = _matmul_t(x_, wr_ref[...]) + bl_ref[...][None, :]

    return pl.pallas_call(
        body,
        grid=(NBLK,),
        in_specs=[_rowspec(), _wspec((D, D)), _wspec((D, D)),
                  _wspec((D, D)), _wspec((D,))],
        out_specs=[_rowspec(), _rowspec(), _rowspec()],
        out_shape=[_SH, _SH, _SH],
    )(x, S0, S1, Wr0, bl0)


def _tc_wr(z, Wr, bl):
    def body(z_ref, wr_ref, bl_ref, w_ref):
        w_ref[...] = _matmul_t(z_ref[...], wr_ref[...]) + bl_ref[...][None, :]

    return pl.pallas_call(
        body,
        grid=(NBLK,),
        in_specs=[_rowspec(), _wspec((D, D)), _wspec((D,))],
        out_specs=_rowspec(),
        out_shape=_SH,
    )(z, Wr, bl)


def _tc_recip(degp):
    def body(dg_ref, recip_ref):
        deg = dg_ref[0, :N] + dg_ref[1, :N]
        recip_ref[...] = (1.0 / jnp.maximum(deg, 1.0))[:, None]

    return pl.pallas_call(
        body, out_shape=jax.ShapeDtypeStruct((N, 1), jnp.float32))(degp)


def _tc_layer(p, recip, w, Wl, lnw, lnb, a, skips, n_out):
    nskip = len(skips)

    def body(*refs):
        (p_ref, r_ref, w_ref, wl_ref,
         lnw_ref, lnb_ref, a_ref) = refs[:7]
        skip_refs = refs[7:7 + nskip]
        out_refs = refs[7 + nskip:7 + nskip + n_out]
        acc_ref, hs_ref = refs[7 + nskip + n_out:]
        i = pl.program_id(0)

        @pl.when(i < NBLK)
        def _():
            agg = (p_ref[0] + p_ref[1]) * r_ref[...]
            h = _matmul_t(agg, wl_ref[...]) + w_ref[...]
            row = pl.multiple_of(i * BLK, BLK)
            hs_ref[pl.ds(row, BLK), :] = h

            @pl.when(i == 0)
            def _():
                acc_ref[0] = 0.0
                acc_ref[1] = 0.0

            acc_ref[0] += jnp.sum(h)
            acc_ref[1] += jnp.sum(h * h)

        @pl.when(i >= NBLK)
        def _():
            cnt = float(N * D)
            m = acc_ref[0] / cnt
            var = acc_ref[1] / cnt - m * m
            std = jnp.sqrt(jnp.maximum(var, 0.0))
            row = pl.multiple_of((i - NBLK) * BLK, BLK)
            h = hs_ref[pl.ds(row, BLK), :]
            hn = ((h - m) / (std + EPS)) * lnw_ref[...][None, :] \
                + lnb_ref[...][None, :]
            act = jnp.where(hn >= 0.0, hn, a_ref[0] * hn)
            if n_out == 2:
                out_refs[0][...] = act
                out_refs[1][...] = act + skip_refs[0][...]
            elif nskip == 2:
                out_refs[0][...] = act + skip_refs[0][...] + skip_refs[1][...]
            else:
                out_refs[0][...] = act

    def _a_idx(i):
        return jnp.minimum(i, NBLK - 1)

    def _b_idx(i):
        return jnp.maximum(i - NBLK, 0)

    in_specs = [
        pl.BlockSpec((NC, BLK, D), lambda i: (0, _a_idx(i), 0)),
        pl.BlockSpec((BLK, 1), lambda i: (_a_idx(i), 0)),
        pl.BlockSpec((BLK, D), lambda i: (_a_idx(i), 0)),
        _wspec((D, D)),
        _wspec((D,)), _wspec((D,)),
        pl.BlockSpec(memory_space=pltpu.SMEM),
    ]
    in_specs += [pl.BlockSpec((BLK, D), lambda i: (_b_idx(i), 0))] * nskip
    return pl.pallas_call(
        body,
        grid=(2 * NBLK,),
        in_specs=in_specs,
        out_specs=[pl.BlockSpec((BLK, D), lambda i: (_b_idx(i), 0))] * n_out,
        out_shape=[_SH] * n_out,
        scratch_shapes=[pltpu.SMEM((2,), jnp.float32),
                        pltpu.VMEM((N, D), jnp.float32)],
    )(p, recip, w, Wl, lnw, lnb, a, *skips)


def kernel(x, edge_index, Wl0, bl0, Wr0, Wl1, bl1, Wr1, Wl2, bl2, Wr2,
           S0, S1, ln0_w, ln0_b, ln1_w, ln1_b, ln2_w, ln2_b, a0, a1, a2):
    npad_e = EPAD - E
    pad_src = jnp.arange(npad_e, dtype=jnp.int32) % NDISC
    pad_dst = N + pad_src
    src2 = jnp.concatenate([edge_index[0], pad_src]).reshape(NROWS, W)
    dst2 = jnp.concatenate([edge_index[1], pad_dst]).reshape(NROWS, W)

    xs0, xs1, w1 = _tc_pre(x, S0, S1, Wr0, bl0)

    p, degp = _sc_agg_call(x, src2, dst2, with_deg=True)
    recip = _tc_recip(degp)
    h1, z2 = _tc_layer(p, recip, w1, Wl0, ln0_w, ln0_b, a0,
                       [xs0], n_out=2)

    (q,) = _sc_agg_call(z2, src2, dst2, with_deg=False)
    w2 = _tc_wr(z2, Wr1, bl1)
    (z3,) = _tc_layer(q, recip, w2, Wl1, ln1_w, ln1_b, a1,
                      [h1, xs1], n_out=1)

    (r,) = _sc_agg_call(z3, src2, dst2, with_deg=False)
    w3 = _tc_wr(z3, Wr2, bl2)
    (ret,) = _tc_layer(r, recip, w3, Wl2, ln2_w, ln2_b, a2,
                       [], n_out=1)
    return ret
